# baseline (speedup 1.0000x reference)
import functools

import jax
import jax.numpy as jnp
from jax import lax
from jax.experimental import pallas as pl
from jax.experimental.pallas import tpu as pltpu, tpu_sc as plsc


def _make_gather(B, M, NC, NS):
    NW = NC * NS
    b_per_w = B // NW
    IDX_W = 128
    n_chunk = b_per_w // IDX_W
    o_per_w = b_per_w // 4
    mesh = plsc.VectorSubcoreMesh(core_axis_name="c", subcore_axis_name="s")

    @functools.partial(
        pl.kernel,
        mesh=mesh,
        out_type=jax.ShapeDtypeStruct((B // 4, 128), jnp.float32),
        scratch_types=[
            pltpu.VMEM((b_per_w,), jnp.int32),
            pltpu.VMEM((b_per_w,), jnp.int32),
            pltpu.VMEM((b_per_w, 128), jnp.float32),
            pltpu.VMEM((o_per_w, 128), jnp.float32),
            pltpu.SemaphoreType.DMA,
        ],
    )
    def k(idx_hbm, tab_hbm, out_hbm, idx_v, midx_v, rows_v, out_v, sem):
        wid = lax.axis_index("s") * NC + lax.axis_index("c")
        base = wid * b_per_w
        pltpu.sync_copy(idx_hbm.at[pl.ds(base, b_per_w)], idx_v)

        def shift_body(t, _):
            v = idx_v[pl.ds(t * 16, 16)]
            midx_v[pl.ds(t * 16, 16)] = lax.shift_right_logical(v, 2)
            return _

        lax.fori_loop(0, b_per_w // 16, shift_body, 0, unroll=4)

        copies = [
            pltpu.async_copy(
                tab_hbm.at[midx_v.at[pl.ds(j * IDX_W, IDX_W)]],
                rows_v.at[pl.ds(j * IDX_W, IDX_W)],
                sem,
            )
            for j in range(n_chunk)
        ]
        for c in copies:
            c.wait()

        def extract_body(g, _):
            j0 = g * 16
            subv = idx_v[pl.ds(j0, 16)] & 3
            for l in range(16):
                j = j0 + l
                sub = subv[l]
                src = rows_v.at[j]
                dst = out_v.at[j // 4]
                s0 = sub * 32
                d0 = (j % 4) * 32
                dst[pl.ds(d0, 16)] = src[pl.ds(s0, 16)]
                dst[pl.ds(d0 + 16, 16)] = src[pl.ds(s0 + 16, 16)]
            return _

        lax.fori_loop(0, b_per_w // 16, extract_body, 0)

        pltpu.sync_copy(out_v, out_hbm.at[pl.ds(wid * o_per_w, o_per_w)])

    return k


def kernel(class_id, table):
    (B,) = class_id.shape
    V, D = table.shape
    info = plsc.get_sparse_core_info()
    NC, NS = info.num_cores, info.num_subcores
    tab = table.reshape(V * D // 128, 128)
    out = _make_gather(B, V * D // 128, NC, NS)(class_id.astype(jnp.int32), tab)
    return out.reshape(B, D)


# speedup vs baseline: 3.6176x; 3.6176x over previous
import functools

import jax
import jax.numpy as jnp
from jax import lax
from jax.experimental import pallas as pl
from jax.experimental.pallas import tpu as pltpu, tpu_sc as plsc


def _make(B, V, NC, NS):
    NW = NC * NS
    b_per_w = B // NW
    BATCH16 = 16
    n_batch = b_per_w // BATCH16
    mesh = plsc.VectorSubcoreMesh(core_axis_name="c", subcore_axis_name="s")

    @functools.partial(
        pl.kernel,
        mesh=mesh,
        out_type=jax.ShapeDtypeStruct((32, B), jnp.float32),
        scratch_types=[
            pltpu.VMEM((b_per_w,), jnp.int32),
            pltpu.VMEM((BATCH16, 32, 128), jnp.float32),
            pltpu.VMEM((32, b_per_w), jnp.float32),
            pltpu.SemaphoreType.DMA,
        ],
        compiler_params=pltpu.CompilerParams(
            disable_bounds_checks=True, needs_layout_passes=False
        ),
    )
    def k(idx_hbm, tab_hbm, out_hbm, idx_v, bufs, stage, sem):
        wid = lax.axis_index("s") * NC + lax.axis_index("c")
        base = wid * b_per_w
        pltpu.sync_copy(idx_hbm.at[pl.ds(base, b_per_w)], idx_v)
        row16 = lax.iota(jnp.int32, 16)

        def batch(b, _):
            j0 = b * BATCH16
            cv = idx_v[pl.ds(j0, BATCH16)]
            copies = []
            for l in range(BATCH16):
                start = pl.multiple_of(cv[l] & ~jnp.int32(127), 128)
                copies.append(
                    pltpu.async_copy(
                        tab_hbm.at[:, pl.ds(start, 128)], bufs.at[l], sem
                    )
                )
            for c in copies:
                c.wait()
            for l in range(BATCH16):
                cl = jnp.full((16,), cv[l] & 127, jnp.int32)
                jv = jnp.full((16,), j0 + l, jnp.int32)
                v0 = plsc.load_gather(bufs.at[l], [row16, cl])
                v1 = plsc.load_gather(bufs.at[l], [row16 + 16, cl])
                plsc.store_scatter(stage, [row16, jv], v0)
                plsc.store_scatter(stage, [row16 + 16, jv], v1)
            return _

        lax.fori_loop(0, n_batch, batch, 0)
        pltpu.sync_copy(stage, out_hbm.at[:, pl.ds(base, b_per_w)])

    return k


def kernel(class_id, table):
    (B,) = class_id.shape
    V, D = table.shape
    info = plsc.get_sparse_core_info()
    NC, NS = info.num_cores, info.num_subcores
    tt = table.T
    out_t = _make(B, V, NC, NS)(class_id.astype(jnp.int32), tt)
    return out_t.T
